# baseline (speedup 1.0000x reference)
import jax
import jax.numpy as jnp
from jax.experimental import pallas as pl
from jax.experimental.pallas import tpu as pltpu


def _chamfer_kernel(t_ref, i_ref, out_ref):
    t = t_ref[0]
    i = i_ref[0]

    eps = jnp.float32(1e-12)
    tn = jnp.sqrt(jnp.sum(t * t, axis=1, keepdims=True))
    t = t / jnp.maximum(tn, eps)
    inn = jnp.sqrt(jnp.sum(i * i, axis=1, keepdims=True))
    i = i / jnp.maximum(inn, eps)

    a2 = jnp.sum(t * t, axis=1, keepdims=True)
    b2 = jax.lax.dot_general(
        jnp.ones((1, i.shape[1]), jnp.float32), i * i,
        (((1,), (1,)), ((), ())), preferred_element_type=jnp.float32)
    ab = jax.lax.dot_general(
        t, i, (((1,), (1,)), ((), ())),
        preferred_element_type=jnp.float32)
    sq = jnp.maximum(a2 + b2 - 2.0 * ab, 0.0)

    K, M = sq.shape
    ridx = jax.lax.broadcasted_iota(jnp.int32, (K, M), 0)
    cidx = jax.lax.broadcasted_iota(jnp.int32, (K, M), 1)
    flat = ridx * M + cidx
    big = jnp.float32(3.4e38)
    bigi = jnp.int32(2**30)

    acc = jnp.float32(0.0)
    for _ in range(4):
        m = jnp.min(sq)
        pos = jnp.min(jnp.where(sq == m, flat, bigi))
        acc = acc + jnp.sqrt(m)
        sq = jnp.where(flat == pos, big, sq)

    @pl.when(pl.program_id(0) == 0)
    def _():
        out_ref[0, 0] = 0.0

    out_ref[0, 0] += acc


def kernel(tokens, interests):
    B, K, D = tokens.shape
    _, M, _ = interests.shape
    total = pl.pallas_call(
        _chamfer_kernel,
        grid=(B,),
        in_specs=[
            pl.BlockSpec((1, K, D), lambda b: (b, 0, 0)),
            pl.BlockSpec((1, M, D), lambda b: (b, 0, 0)),
        ],
        out_specs=pl.BlockSpec(memory_space=pltpu.SMEM),
        out_shape=jax.ShapeDtypeStruct((1, 1), jnp.float32),
    )(tokens, interests)
    return (total[0, 0] / (B * 4)).astype(jnp.float32)


# speedup vs baseline: 1.4433x; 1.4433x over previous
import jax
import jax.numpy as jnp
from jax.experimental import pallas as pl
from jax.experimental.pallas import tpu as pltpu

_PAIR = 2
_BB = 8


def _sq_dists(t, i):
    eps = jnp.float32(1e-12)
    nt2 = jnp.sum(t * t, axis=1, keepdims=True)
    na = jnp.sqrt(nt2)
    ra = 1.0 / jnp.maximum(na, eps)
    a2 = (na * ra) ** 2
    ii = i * i
    ni2 = jax.lax.dot_general(
        jnp.ones((1, i.shape[1]), jnp.float32), ii,
        (((1,), (1,)), ((), ())), preferred_element_type=jnp.float32)
    nb = jnp.sqrt(ni2)
    rb = 1.0 / jnp.maximum(nb, eps)
    b2 = (nb * rb) ** 2
    ab = jax.lax.dot_general(
        t, i, (((1,), (1,)), ((), ())),
        preferred_element_type=jnp.float32)
    return jnp.maximum(a2 + b2 - 2.0 * (ab * ra * rb), 0.0)


def _chamfer_kernel(t_ref, i_ref, out_ref):
    K = t_ref.shape[1]
    M = i_ref.shape[1]
    W = _PAIR * M
    nchunks = K // 8
    big = jnp.float32(3.0e38)
    bigi = jnp.int32(2**30)

    acc = jnp.float32(0.0)
    for p in range(_BB // _PAIR):
        sqs = [_sq_dists(t_ref[_PAIR * p + j], i_ref[_PAIR * p + j])
               for j in range(_PAIR)]
        sq = jnp.concatenate(sqs, axis=1)

        a0 = jax.lax.slice(sq, (0, 0), (8, W))
        a1 = jnp.full((8, W), big)
        a2_ = jnp.full((8, W), big)
        a3 = jnp.full((8, W), big)
        for c in range(1, nchunks):
            v = jax.lax.slice(sq, (8 * c, 0), (8 * c + 8, W))
            lo = jnp.minimum(a0, v); v = jnp.maximum(a0, v); a0 = lo
            lo = jnp.minimum(a1, v); v = jnp.maximum(a1, v); a1 = lo
            lo = jnp.minimum(a2_, v); v = jnp.maximum(a2_, v); a2_ = lo
            a3 = jnp.minimum(a3, v)
        cand = jnp.concatenate([a0, a1, a2_, a3], axis=0)

        lane = jax.lax.broadcasted_iota(jnp.int32, (32, W), 1)
        fi = jax.lax.broadcasted_iota(jnp.int32, (32, W), 0) * W + lane
        for j in range(_PAIR):
            in_b = jnp.logical_and(lane >= j * M, lane < (j + 1) * M)
            c_b = jnp.where(in_b, cand, big)
            for _ in range(4):
                m = jnp.min(c_b)
                pos = jnp.min(jnp.where(c_b == m, fi, bigi))
                acc = acc + jnp.sqrt(m)
                c_b = jnp.where(fi == pos, big, c_b)

    @pl.when(pl.program_id(0) == 0)
    def _():
        out_ref[0, 0] = 0.0

    out_ref[0, 0] += acc


def kernel(tokens, interests):
    B, K, D = tokens.shape
    _, M, _ = interests.shape
    total = pl.pallas_call(
        _chamfer_kernel,
        grid=(B // _BB,),
        in_specs=[
            pl.BlockSpec((_BB, K, D), lambda b: (b, 0, 0)),
            pl.BlockSpec((_BB, M, D), lambda b: (b, 0, 0)),
        ],
        out_specs=pl.BlockSpec(memory_space=pltpu.SMEM),
        out_shape=jax.ShapeDtypeStruct((1, 1), jnp.float32),
    )(tokens, interests)
    return (total[0, 0] / (B * 4)).astype(jnp.float32)


# speedup vs baseline: 8.7126x; 6.0367x over previous
import jax
import jax.numpy as jnp
from jax.experimental import pallas as pl
from jax.experimental.pallas import tpu as pltpu

_PAIR = 2
_BB = 8


def _sq_dists(t, i):
    eps = jnp.float32(1e-12)
    nt2 = jnp.sum(t * t, axis=1, keepdims=True)
    na = jnp.sqrt(nt2)
    ra = 1.0 / jnp.maximum(na, eps)
    a2 = (na * ra) ** 2
    ii = i * i
    ni2 = jax.lax.dot_general(
        jnp.ones((1, i.shape[1]), jnp.float32), ii,
        (((1,), (1,)), ((), ())), preferred_element_type=jnp.float32)
    nb = jnp.sqrt(ni2)
    rb = 1.0 / jnp.maximum(nb, eps)
    b2 = (nb * rb) ** 2
    ab = jax.lax.dot_general(
        t, i, (((1,), (1,)), ((), ())),
        preferred_element_type=jnp.float32)
    return jnp.maximum(a2 + b2 - 2.0 * (ab * ra * rb), 0.0)


def _insert(q, v):
    a0, a1, a2, a3 = q
    lo = jnp.minimum(a0, v); v = jnp.maximum(a0, v); a0 = lo
    lo = jnp.minimum(a1, v); v = jnp.maximum(a1, v); a1 = lo
    lo = jnp.minimum(a2, v); v = jnp.maximum(a2, v); a2 = lo
    a3 = jnp.minimum(a3, v)
    return (a0, a1, a2, a3)


def _merge4(a, b, sort=True):
    c0 = jnp.minimum(a[0], b[3])
    c1 = jnp.minimum(a[1], b[2])
    c2 = jnp.minimum(a[2], b[1])
    c3 = jnp.minimum(a[3], b[0])
    if not sort:
        return (c0, c1, c2, c3)
    lo02 = jnp.minimum(c0, c2); hi02 = jnp.maximum(c0, c2)
    lo13 = jnp.minimum(c1, c3); hi13 = jnp.maximum(c1, c3)
    return (jnp.minimum(lo02, lo13), jnp.maximum(lo02, lo13),
            jnp.minimum(hi02, hi13), jnp.maximum(hi02, hi13))


def _chamfer_kernel(t_ref, i_ref, out_ref, vacc_ref):
    K = t_ref.shape[1]
    M = i_ref.shape[1]
    W = _PAIR * M
    nchunks = K // 8
    half = nchunks // 2
    big = jnp.float32(3.0e38)

    li = jax.lax.broadcasted_iota(jnp.int32, (8, W), 1)
    si = jax.lax.broadcasted_iota(jnp.int32, (8, W), 0)
    outmask = jnp.logical_and(si == 0, li % M == 0)

    @pl.when(pl.program_id(0) == 0)
    def _():
        vacc_ref[...] = jnp.zeros((8, W), jnp.float32)

    acc = jnp.zeros((8, W), jnp.float32)
    for p in range(_BB // _PAIR):
        sqs = [_sq_dists(t_ref[_PAIR * p + j], i_ref[_PAIR * p + j])
               for j in range(_PAIR)]
        sq = jnp.concatenate(sqs, axis=1)

        def chunk(c):
            return jax.lax.slice(sq, (8 * c, 0), (8 * c + 8, W))

        qa = (chunk(0), jnp.full((8, W), big), jnp.full((8, W), big),
              jnp.full((8, W), big))
        qb = (chunk(half), jnp.full((8, W), big), jnp.full((8, W), big),
              jnp.full((8, W), big))
        for c in range(1, half):
            qa = _insert(qa, chunk(c))
        for c in range(half + 1, nchunks):
            qb = _insert(qb, chunk(c))
        q = _merge4(qa, qb)

        for s in (4, 2, 1):
            r = tuple(jnp.roll(x, -s, axis=0) for x in q)
            q = _merge4(q, r)
        lane_shifts = []
        s = M // 2
        while s >= 1:
            lane_shifts.append(s)
            s //= 2
        for idx, s in enumerate(lane_shifts):
            r = tuple(jnp.roll(x, -s, axis=1) for x in q)
            q = _merge4(q, r, sort=(idx < len(lane_shifts) - 1))

        d = jnp.sqrt(q[0]) + jnp.sqrt(q[1]) + jnp.sqrt(q[2]) + jnp.sqrt(q[3])
        acc = acc + jnp.where(outmask, d, 0.0)

    vacc_ref[...] += acc

    @pl.when(pl.program_id(0) == pl.num_programs(0) - 1)
    def _():
        out_ref[0, 0] = jnp.sum(vacc_ref[...])


def kernel(tokens, interests):
    B, K, D = tokens.shape
    _, M, _ = interests.shape
    total = pl.pallas_call(
        _chamfer_kernel,
        grid=(B // _BB,),
        in_specs=[
            pl.BlockSpec((_BB, K, D), lambda b: (b, 0, 0)),
            pl.BlockSpec((_BB, M, D), lambda b: (b, 0, 0)),
        ],
        out_specs=pl.BlockSpec(memory_space=pltpu.SMEM),
        out_shape=jax.ShapeDtypeStruct((1, 1), jnp.float32),
        scratch_shapes=[pltpu.VMEM((8, _PAIR * M), jnp.float32)],
    )(tokens, interests)
    return (total[0, 0] / (B * 4)).astype(jnp.float32)


# speedup vs baseline: 9.8470x; 1.1302x over previous
import functools

import jax
import jax.numpy as jnp
from jax.experimental import pallas as pl
from jax.experimental.pallas import tpu as pltpu
from jax.experimental.pallas import tpu_sc as plsc

_PAIR = 2
_BB = 16


def _dot(a, b):
    return jax.lax.dot_general(a, b, (((1,), (1,)), ((), ())),
                               preferred_element_type=jnp.float32)


def _sq_dists(t, i):
    eps2 = jnp.float32(1e-24)
    ones_row = jnp.ones((1, t.shape[1]), jnp.float32)
    nt2 = _dot(t * t, ones_row)
    ra = jax.lax.rsqrt(jnp.maximum(nt2, eps2))
    a2 = nt2 * ra * ra
    ni2 = _dot(ones_row, i * i)
    rb = jax.lax.rsqrt(jnp.maximum(ni2, eps2))
    b2 = ni2 * rb * rb
    ab = _dot(t, i)
    return jnp.maximum(a2 + b2 - 2.0 * (ab * ra * rb), 0.0)


def _insert(q, v):
    a0, a1, a2, a3 = q
    lo = jnp.minimum(a0, v); v = jnp.maximum(a0, v); a0 = lo
    lo = jnp.minimum(a1, v); v = jnp.maximum(a1, v); a1 = lo
    lo = jnp.minimum(a2, v); v = jnp.maximum(a2, v); a2 = lo
    a3 = jnp.minimum(a3, v)
    return (a0, a1, a2, a3)


def _sort_bitonic4(c0, c1, c2, c3):
    lo02 = jnp.minimum(c0, c2); hi02 = jnp.maximum(c0, c2)
    lo13 = jnp.minimum(c1, c3); hi13 = jnp.maximum(c1, c3)
    return (jnp.minimum(lo02, lo13), jnp.maximum(lo02, lo13),
            jnp.minimum(hi02, hi13), jnp.maximum(hi02, hi13))


def _merge4(a, b):
    c0 = jnp.minimum(a[0], b[3])
    c1 = jnp.minimum(a[1], b[2])
    c2 = jnp.minimum(a[2], b[1])
    c3 = jnp.minimum(a[3], b[0])
    return _sort_bitonic4(c0, c1, c2, c3)


def _insert_pair(q, v1, v2):
    w = jnp.minimum(v1, v2)
    z = jnp.maximum(v1, v2)
    c2 = jnp.minimum(q[2], z)
    c3 = jnp.minimum(q[3], w)
    return _sort_bitonic4(q[0], q[1], c2, c3)


def _cand_kernel(t_ref, i_ref, out_ref):
    K = t_ref.shape[1]
    M = i_ref.shape[1]
    W = _PAIR * M
    nchunks = K // 8
    half = nchunks // 2
    big = jnp.float32(3.0e38)

    for p in range(_BB // _PAIR):
        sqs = [_sq_dists(t_ref[_PAIR * p + j], i_ref[_PAIR * p + j])
               for j in range(_PAIR)]
        sq = jnp.concatenate(sqs, axis=1)

        def chunk(c):
            return jax.lax.slice(sq, (8 * c, 0), (8 * c + 8, W))

        def top4_of(chunks):
            w = jnp.minimum(chunk(chunks[0]), chunk(chunks[1]))
            z = jnp.maximum(chunk(chunks[0]), chunk(chunks[1]))
            q = (w, z, jnp.full((8, W), big), jnp.full((8, W), big))
            rest = chunks[2:]
            for k in range(0, len(rest) - 1, 2):
                q = _insert_pair(q, chunk(rest[k]), chunk(rest[k + 1]))
            if len(rest) % 2:
                q = _insert(q, chunk(rest[-1]))
            return q

        qa = top4_of(list(range(0, half)))
        qb = top4_of(list(range(half, nchunks)))
        q = _merge4(qa, qb)

        for s in (4, 2, 1):
            r = tuple(jnp.roll(x, -s, axis=0) for x in q)
            q = _merge4(q, r)

        for j in range(4):
            out_ref[p, j:j + 1, :] = jnp.sqrt(
                jax.lax.slice(q[j], (0, 0), (1, W)))


def _lowest16(a, b):
    return jax.lax.sort(jnp.minimum(a, jax.lax.rev(b, (0,))))


def _sc_topk_kernel(cand_hbm, out_hbm, buf, obuf):
    info = plsc.get_sparse_core_info()
    nc = info.num_cores
    nw = nc * info.num_subcores
    wid = jax.lax.axis_index("s") * nc + jax.lax.axis_index("c")
    npairs = cand_hbm.shape[0]
    per_w = npairs // nw
    lanes_per_batch = cand_hbm.shape[2] // _PAIR
    mask4 = jax.lax.iota(jnp.int32, 16) < 4

    def body(pp, acc):
        p = wid * per_w + pp
        pltpu.sync_copy(cand_hbm.at[p], buf)
        for h in range(_PAIR):
            qs = []
            for j in range(4):
                qs.append([buf[j, pl.ds(h * lanes_per_batch + o * 16, 16)]
                           for o in range(lanes_per_batch // 16)])
            q = (qs[0][0], qs[1][0], qs[2][0], qs[3][0])
            for o in range(1, len(qs[0])):
                q = _merge4(q, (qs[0][o], qs[1][o], qs[2][o], qs[3][o]))
            u = _lowest16(jax.lax.sort(q[0]), jax.lax.sort(q[1]))
            v = _lowest16(jax.lax.sort(q[2]), jax.lax.sort(q[3]))
            w = _lowest16(u, v)
            acc = acc + jnp.where(mask4, w, 0.0)
        return acc

    acc = jax.lax.fori_loop(0, per_w, body, jnp.zeros((16,), jnp.float32))
    obuf[...] = acc
    pltpu.sync_copy(obuf, out_hbm.at[wid])


def _sc_topk(cand):
    npairs = cand.shape[0]
    mesh = plsc.VectorSubcoreMesh(core_axis_name="c", subcore_axis_name="s")
    nw = mesh.num_cores * mesh.num_subcores
    assert npairs % nw == 0
    return pl.kernel(
        _sc_topk_kernel,
        out_type=jax.ShapeDtypeStruct((nw, 16), jnp.float32),
        mesh=mesh,
        compiler_params=pltpu.CompilerParams(needs_layout_passes=False),
        scratch_types=[
            pltpu.VMEM((4, cand.shape[2]), jnp.float32),
            pltpu.VMEM((16,), jnp.float32),
        ],
    )(cand)


def kernel(tokens, interests):
    B, K, D = tokens.shape
    _, M, _ = interests.shape
    cand = pl.pallas_call(
        _cand_kernel,
        grid=(B // _BB,),
        in_specs=[
            pl.BlockSpec((_BB, K, D), lambda b: (b, 0, 0)),
            pl.BlockSpec((_BB, M, D), lambda b: (b, 0, 0)),
        ],
        out_specs=pl.BlockSpec((_BB // _PAIR, 4, _PAIR * M),
                               lambda b: (b, 0, 0)),
        out_shape=jax.ShapeDtypeStruct((B // _PAIR, 4, _PAIR * M),
                                       jnp.float32),
    )(tokens, interests)
    partials = _sc_topk(cand)
    return (jnp.sum(partials) / (B * 4)).astype(jnp.float32)


# speedup vs baseline: 12.5277x; 1.2722x over previous
import functools

import jax
import jax.numpy as jnp
from jax.experimental import pallas as pl
from jax.experimental.pallas import tpu as pltpu
from jax.experimental.pallas import tpu_sc as plsc

_PAIR = 2
_BB = 16


def _dot(a, b):
    return jax.lax.dot_general(a, b, (((1,), (1,)), ((), ())),
                               preferred_element_type=jnp.float32)


def _sq_dists(t, i):
    eps2 = jnp.float32(1e-24)
    K = t.shape[0]
    M = i.shape[0]
    ones_row = jnp.ones((1, t.shape[1]), jnp.float32)
    nt2 = _dot(t * t, ones_row)
    ra = jax.lax.rsqrt(jnp.maximum(nt2, eps2))
    a2 = nt2 * ra * ra
    ni2 = _dot(i * i, ones_row)
    rb = jax.lax.rsqrt(jnp.maximum(ni2, eps2))
    b2 = ni2 * rb * rb
    i2 = i * (-2.0 * rb)
    x = _dot(t, i2)
    pa = jnp.concatenate([a2, jnp.ones((K, 1), jnp.float32)], axis=1)
    pb = jnp.concatenate([jnp.ones((M, 1), jnp.float32), b2], axis=1)
    p = _dot(pa, pb)
    return jnp.maximum(x * ra + p, 0.0)


def _insert(q, v):
    a0, a1, a2, a3 = q
    lo = jnp.minimum(a0, v); v = jnp.maximum(a0, v); a0 = lo
    lo = jnp.minimum(a1, v); v = jnp.maximum(a1, v); a1 = lo
    lo = jnp.minimum(a2, v); v = jnp.maximum(a2, v); a2 = lo
    a3 = jnp.minimum(a3, v)
    return (a0, a1, a2, a3)


def _sort_bitonic4(c0, c1, c2, c3):
    lo02 = jnp.minimum(c0, c2); hi02 = jnp.maximum(c0, c2)
    lo13 = jnp.minimum(c1, c3); hi13 = jnp.maximum(c1, c3)
    return (jnp.minimum(lo02, lo13), jnp.maximum(lo02, lo13),
            jnp.minimum(hi02, hi13), jnp.maximum(hi02, hi13))


def _merge4(a, b):
    c0 = jnp.minimum(a[0], b[3])
    c1 = jnp.minimum(a[1], b[2])
    c2 = jnp.minimum(a[2], b[1])
    c3 = jnp.minimum(a[3], b[0])
    return _sort_bitonic4(c0, c1, c2, c3)


def _insert_pair(q, v1, v2):
    w = jnp.minimum(v1, v2)
    z = jnp.maximum(v1, v2)
    c2 = jnp.minimum(q[2], z)
    c3 = jnp.minimum(q[3], w)
    return _sort_bitonic4(q[0], q[1], c2, c3)


def _cand_kernel(t_ref, i_ref, out_ref):
    K = t_ref.shape[1]
    M = i_ref.shape[1]
    W = _PAIR * M
    nchunks = K // 8
    half = nchunks // 2
    big = jnp.float32(3.0e38)

    for p in range(_BB // _PAIR):
        sqs = [_sq_dists(t_ref[_PAIR * p + j], i_ref[_PAIR * p + j])
               for j in range(_PAIR)]
        sq = jnp.concatenate(sqs, axis=1)

        def chunk(c):
            return jax.lax.slice(sq, (8 * c, 0), (8 * c + 8, W))

        def top4_of(chunks):
            w = jnp.minimum(chunk(chunks[0]), chunk(chunks[1]))
            z = jnp.maximum(chunk(chunks[0]), chunk(chunks[1]))
            q = (w, z, jnp.full((8, W), big), jnp.full((8, W), big))
            rest = chunks[2:]
            for k in range(0, len(rest) - 1, 2):
                q = _insert_pair(q, chunk(rest[k]), chunk(rest[k + 1]))
            if len(rest) % 2:
                q = _insert(q, chunk(rest[-1]))
            return q

        qa = top4_of(list(range(0, half)))
        qb = top4_of(list(range(half, nchunks)))
        q = _merge4(qa, qb)

        for s in (4, 2, 1):
            r = tuple(jnp.roll(x, -s, axis=0) for x in q)
            q = _merge4(q, r)

        for j in range(4):
            out_ref[p, j:j + 1, :] = jnp.sqrt(
                jax.lax.slice(q[j], (0, 0), (1, W)))


def _lowest16(a, b):
    return jax.lax.sort(jnp.minimum(a, jax.lax.rev(b, (0,))))


def _sc_topk_kernel(cand_hbm, out_hbm, buf, obuf):
    info = plsc.get_sparse_core_info()
    nc = info.num_cores
    nw = nc * info.num_subcores
    wid = jax.lax.axis_index("s") * nc + jax.lax.axis_index("c")
    npairs = cand_hbm.shape[0]
    per_w = npairs // nw
    lanes_per_batch = cand_hbm.shape[2] // _PAIR
    mask4 = jax.lax.iota(jnp.int32, 16) < 4

    def body(pp, acc):
        p = wid * per_w + pp
        pltpu.sync_copy(cand_hbm.at[p], buf)
        for h in range(_PAIR):
            qs = []
            for j in range(4):
                qs.append([buf[j, pl.ds(h * lanes_per_batch + o * 16, 16)]
                           for o in range(lanes_per_batch // 16)])
            q = (qs[0][0], qs[1][0], qs[2][0], qs[3][0])
            for o in range(1, len(qs[0])):
                q = _merge4(q, (qs[0][o], qs[1][o], qs[2][o], qs[3][o]))
            u = _lowest16(jax.lax.sort(q[0]), jax.lax.sort(q[1]))
            v = _lowest16(jax.lax.sort(q[2]), jax.lax.sort(q[3]))
            w = _lowest16(u, v)
            acc = acc + jnp.where(mask4, w, 0.0)
        return acc

    acc = jax.lax.fori_loop(0, per_w, body, jnp.zeros((16,), jnp.float32))
    obuf[...] = acc
    pltpu.sync_copy(obuf, out_hbm.at[wid])


def _sc_topk(cand):
    npairs = cand.shape[0]
    mesh = plsc.VectorSubcoreMesh(core_axis_name="c", subcore_axis_name="s")
    nw = mesh.num_cores * mesh.num_subcores
    assert npairs % nw == 0
    return pl.kernel(
        _sc_topk_kernel,
        out_type=jax.ShapeDtypeStruct((nw, 16), jnp.float32),
        mesh=mesh,
        compiler_params=pltpu.CompilerParams(needs_layout_passes=False),
        scratch_types=[
            pltpu.VMEM((4, cand.shape[2]), jnp.float32),
            pltpu.VMEM((16,), jnp.float32),
        ],
    )(cand)


def kernel(tokens, interests):
    B, K, D = tokens.shape
    _, M, _ = interests.shape
    cand = pl.pallas_call(
        _cand_kernel,
        grid=(B // _BB,),
        in_specs=[
            pl.BlockSpec((_BB, K, D), lambda b: (b, 0, 0)),
            pl.BlockSpec((_BB, M, D), lambda b: (b, 0, 0)),
        ],
        out_specs=pl.BlockSpec((_BB // _PAIR, 4, _PAIR * M),
                               lambda b: (b, 0, 0)),
        out_shape=jax.ShapeDtypeStruct((B // _PAIR, 4, _PAIR * M),
                                       jnp.float32),
    )(tokens, interests)
    partials = _sc_topk(cand)
    return (jnp.sum(partials) / (B * 4)).astype(jnp.float32)
